# baseline (speedup 1.0000x reference)
import functools

import jax
import jax.numpy as jnp
from jax import lax
from jax.experimental import pallas as pl
from jax.experimental.pallas import tpu as pltpu
from jax.experimental.pallas import tpu_sc as plsc

_NC = 2
_NS = 16
_L = 16


@functools.partial(jax.jit, static_argnums=())
def _embed_stem(idx_flat, tok_emb, pos):
    BT = idx_flat.shape[0]
    T, D = pos.shape
    B = BT // T
    NW = _NC * _NS
    TW = T // NW

    mesh = plsc.VectorSubcoreMesh(core_axis_name="c", subcore_axis_name="s")

    @functools.partial(
        pl.kernel,
        mesh=mesh,
        out_type=jax.ShapeDtypeStruct((BT, D), jnp.float32),
        scratch_types=[
            pltpu.VMEM((TW,), jnp.int32),
            pltpu.VMEM((TW, D), jnp.float32),
            pltpu.VMEM((TW, D), jnp.float32),
            pltpu.SemaphoreType.DMA,
        ],
    )
    def k(idx_hbm, tab_hbm, pos_hbm, out_hbm, idx_v, pos_v, rows_v, sem):
        wid = lax.axis_index("s") * _NC + lax.axis_index("c")
        t0 = wid * TW
        pltpu.sync_copy(pos_hbm.at[pl.ds(t0, TW)], pos_v)
        for b in range(B):
            base = b * T + t0
            pltpu.sync_copy(idx_hbm.at[pl.ds(base, TW)], idx_v)
            pltpu.async_copy(tab_hbm.at[idx_v], rows_v, sem).wait()

            def row_add(r, _):
                for c in range(D // _L):
                    sl = pl.ds(c * _L, _L)
                    rows_v[r, sl] = rows_v[r, sl] + pos_v[r, sl]
                return 0

            lax.fori_loop(0, TW, row_add, 0)
            pltpu.sync_copy(rows_v, out_hbm.at[pl.ds(base, TW)])

    return k(idx_flat, tok_emb, pos)


def kernel(idx, tok_emb, pos_embed):
    b, t = idx.shape
    d = tok_emb.shape[1]
    idx_flat = idx.reshape(-1).astype(jnp.int32)
    pos = pos_embed[0, :t, :]
    out = _embed_stem(idx_flat, tok_emb, pos)
    return out.reshape(b, t, d)
